# Initial kernel scaffold; baseline (speedup 1.0000x reference)
import functools

import jax
import jax.numpy as jnp
from jax import lax
from jax.experimental import pallas as pl
from jax.experimental.pallas import tpu as pltpu
from jax.experimental.pallas import tpu_sc as plsc

DAY_LEN = 288
D_MODEL = 64

NC = 2
NS = 16
L = 16
NW = NC * NS

CHUNK = 128


def _sc_lookup(n_total: int):
  per_w = n_total // NW
  steps = per_w // CHUNK
  mesh = plsc.VectorSubcoreMesh(
      core_axis_name="c", subcore_axis_name="s", num_cores=NC,
      num_subcores=NS)

  @functools.partial(
      pl.kernel,
      mesh=mesh,
      out_type=jax.ShapeDtypeStruct((n_total, D_MODEL), jnp.float32),
      scratch_types=[
          pltpu.VMEM((CHUNK,), jnp.float32),
          pltpu.VMEM((CHUNK,), jnp.int32),
          pltpu.VMEM((CHUNK, D_MODEL), jnp.float32),
          pltpu.SemaphoreType.DMA,
      ],
  )
  def k(x_hbm, table_hbm, out_hbm, x_v, idx_v, rows_v, sem):
    wid = lax.axis_index("s") * NC + lax.axis_index("c")
    base = wid * per_w

    def step(g, carry):
      off = base + g * CHUNK
      pltpu.sync_copy(x_hbm.at[pl.ds(off, CHUNK)], x_v)
      for j in range(CHUNK // L):
        xv = x_v[pl.ds(j * L, L)]
        iv = jnp.clip((xv * float(DAY_LEN)).astype(jnp.int32), 0, DAY_LEN - 1)
        idx_v[pl.ds(j * L, L)] = iv
      cp = pltpu.make_async_copy(table_hbm.at[idx_v], rows_v, sem)
      cp.start()
      cp.wait()
      pltpu.sync_copy(rows_v, out_hbm.at[pl.ds(off, CHUNK), :])
      return carry

    lax.fori_loop(0, steps, step, 0)

  return k


def kernel(x_time_norm, day_emb):
  n_total = x_time_norm.shape[0] * x_time_norm.shape[1]
  out = _sc_lookup(n_total)(x_time_norm.reshape(n_total), day_emb)
  return out.reshape(x_time_norm.shape[0], x_time_norm.shape[1], D_MODEL)


# speedup vs baseline: 3.2869x; 3.2869x over previous
import functools

import jax
import jax.numpy as jnp
from jax import lax
from jax.experimental import pallas as pl
from jax.experimental.pallas import tpu as pltpu
from jax.experimental.pallas import tpu_sc as plsc

DAY_LEN = 288
D_MODEL = 64

NC = 2
NS = 16
L = 16
NW = NC * NS

CHUNK = 128


def _sc_lookup(n_total: int):
  per_w = n_total // NW
  steps = per_w // CHUNK
  mesh = plsc.VectorSubcoreMesh(
      core_axis_name="c", subcore_axis_name="s", num_cores=NC,
      num_subcores=NS)

  @functools.partial(
      pl.kernel,
      mesh=mesh,
      compiler_params=pltpu.CompilerParams(use_tc_tiling_on_sc=False),
      out_type=jax.ShapeDtypeStruct((n_total, D_MODEL), jnp.float32),
      scratch_types=[
          pltpu.VMEM((CHUNK,), jnp.float32),
          pltpu.VMEM((CHUNK,), jnp.int32),
          pltpu.VMEM((CHUNK, D_MODEL), jnp.float32),
          pltpu.SemaphoreType.DMA,
      ],
  )
  def k(x_hbm, table_hbm, out_hbm, x_v, idx_v, rows_v, sem):
    wid = lax.axis_index("s") * NC + lax.axis_index("c")
    base = wid * per_w

    def step(g, carry):
      off = base + g * CHUNK
      pltpu.sync_copy(x_hbm.at[pl.ds(off, CHUNK)], x_v)
      for j in range(CHUNK // L):
        xv = x_v[pl.ds(j * L, L)]
        iv = jnp.clip((xv * float(DAY_LEN)).astype(jnp.int32), 0, DAY_LEN - 1)
        idx_v[pl.ds(j * L, L)] = iv
      cp = pltpu.make_async_copy(table_hbm.at[idx_v], rows_v, sem)
      cp.start()
      cp.wait()
      pltpu.sync_copy(rows_v, out_hbm.at[pl.ds(off, CHUNK), :])
      return carry

    lax.fori_loop(0, steps, step, 0)

  return k


def kernel(x_time_norm, day_emb):
  n_total = x_time_norm.shape[0] * x_time_norm.shape[1]
  out = _sc_lookup(n_total)(x_time_norm.reshape(n_total), day_emb)
  return out.reshape(x_time_norm.shape[0], x_time_norm.shape[1], D_MODEL)


# speedup vs baseline: 3.4065x; 1.0364x over previous
import functools

import jax
import jax.numpy as jnp
from jax import lax
from jax.experimental import pallas as pl
from jax.experimental.pallas import tpu as pltpu
from jax.experimental.pallas import tpu_sc as plsc

DAY_LEN = 288
D_MODEL = 64

NC = 2
NS = 16
L = 16
NW = NC * NS

CHUNK = 128
NB = 8
XBC = NB * CHUNK


def _sc_lookup(n_total: int):
  per_w = n_total // NW
  steps = per_w // CHUNK
  t_outer = steps // NB
  mesh = plsc.VectorSubcoreMesh(
      core_axis_name="c", subcore_axis_name="s", num_cores=NC,
      num_subcores=NS)

  @functools.partial(
      pl.kernel,
      mesh=mesh,
      compiler_params=pltpu.CompilerParams(use_tc_tiling_on_sc=False),
      out_type=jax.ShapeDtypeStruct((n_total, D_MODEL), jnp.float32),
      scratch_types=(
          [
              pltpu.VMEM((2 * XBC,), jnp.float32),
              pltpu.VMEM((NB, CHUNK), jnp.int32),
              pltpu.VMEM((NB, CHUNK, D_MODEL), jnp.float32),
          ]
          + [pltpu.SemaphoreType.DMA] * (2 + NB + NB)
      ),
  )
  def k(x_hbm, table_hbm, out_hbm, x_v, idx_v, rows_v, *sems):
    sem_x = sems[0:2]
    sem_g = sems[2:2 + NB]
    sem_o = sems[2 + NB:2 + 2 * NB]
    wid = lax.axis_index("s") * NC + lax.axis_index("c")
    base = wid * per_w

    def x_copy(t, xb_static):
      return pltpu.make_async_copy(
          x_hbm.at[pl.ds(base + t * XBC, XBC)],
          x_v.at[pl.ds(xb_static * XBC, XBC)], sem_x[xb_static])

    x_copy(0, 0).start()

    def outer(t, carry):
      xb = lax.rem(t, 2)
      xoff = xb * XBC

      @pl.when(xb == 0)
      def _():
        x_copy(t, 0).wait()

      @pl.when(xb == 1)
      def _():
        x_copy(t, 1).wait()

      @pl.when(jnp.logical_and(xb == 0, t + 1 < t_outer))
      def _():
        x_copy(t + 1, 1).start()

      @pl.when(jnp.logical_and(xb == 1, t + 1 < t_outer))
      def _():
        x_copy(t + 1, 0).start()

      for b in range(NB):
        for j in range(CHUNK // L):
          xv = x_v[pl.ds(xoff + b * CHUNK + j * L, L)]
          iv = jnp.clip((xv * float(DAY_LEN)).astype(jnp.int32), 0,
                        DAY_LEN - 1)
          idx_v[b, pl.ds(j * L, L)] = iv

        @pl.when(t > 0)
        def _(b=b):
          pltpu.make_async_copy(
              rows_v.at[b],
              out_hbm.at[pl.ds(base + ((t - 1) * NB + b) * CHUNK, CHUNK), :],
              sem_o[b]).wait()

        pltpu.make_async_copy(table_hbm.at[idx_v.at[b]], rows_v.at[b],
                              sem_g[b]).start()

      for b in range(NB):
        pltpu.make_async_copy(table_hbm.at[idx_v.at[b]], rows_v.at[b],
                              sem_g[b]).wait()
        pltpu.make_async_copy(
            rows_v.at[b],
            out_hbm.at[pl.ds(base + (t * NB + b) * CHUNK, CHUNK), :],
            sem_o[b]).start()
      return carry

    lax.fori_loop(0, t_outer, outer, 0)

    for b in range(NB):
      pltpu.make_async_copy(
          rows_v.at[b],
          out_hbm.at[pl.ds(base + ((t_outer - 1) * NB + b) * CHUNK, CHUNK), :],
          sem_o[b]).wait()

  return k


def kernel(x_time_norm, day_emb):
  n_total = x_time_norm.shape[0] * x_time_norm.shape[1]
  out = _sc_lookup(n_total)(x_time_norm.reshape(n_total), day_emb)
  return out.reshape(x_time_norm.shape[0], x_time_norm.shape[1], D_MODEL)
